# Initial kernel scaffold; baseline (speedup 1.0000x reference)
import jax
import jax.numpy as jnp
from jax.experimental import pallas as pl


def kernel(frame_feature, Wg_d, bg_d, Wg_1, bg_1, Wg_o, bg_o, Wl_d, bl_d, Wl_1, bl_1, Wl_o, bl_o):
    raise NotImplementedError("write your pallas kernel here")


# speedup vs baseline: 1.0192x; 1.0192x over previous
import functools

import jax
import jax.numpy as jnp
from jax.experimental import pallas as pl

_NW = 64
_NTOKEN = 128
_TILE = 512


def _heads_kernel(x0_ref, x1_ref, x2_ref, w0_ref, w1_ref, w2_ref,
                  wxg_ref, wxl_ref, mg_ref, ml_ref, cg_ref, cl_ref,
                  g_ref, l_ref):
    acc = jnp.dot(x0_ref[...], w0_ref[...], preferred_element_type=jnp.float32)
    acc = acc + jnp.dot(x1_ref[...], w1_ref[...], preferred_element_type=jnp.float32)
    acc = acc + jnp.dot(x2_ref[...], w2_ref[...], preferred_element_type=jnp.float32)
    r = jnp.maximum(acc, 0.0).astype(jnp.bfloat16)
    rg = r[:, :1024]
    rl = r[:, 1024:]
    x1b = x1_ref[...]
    g = jnp.dot(x1b, wxg_ref[...], preferred_element_type=jnp.float32)
    g = g + jnp.dot(rg, mg_ref[...], preferred_element_type=jnp.float32)
    g_ref[...] = g + cg_ref[...]
    l = jnp.dot(x1b, wxl_ref[...], preferred_element_type=jnp.float32)
    l = l + jnp.dot(rl, ml_ref[...], preferred_element_type=jnp.float32)
    l_ref[...] = l + cl_ref[...]


def _run_heads(x0, x1, x2, w0, w1, w2, wxg, wxl, mg, ml, cg, cl, T):
    grid = (T // _TILE,)
    xspec = pl.BlockSpec((_TILE, 1024), lambda i: (i, 0))

    def fullspec(shape):
        return pl.BlockSpec(shape, lambda i: tuple(0 for _ in shape))

    return pl.pallas_call(
        _heads_kernel,
        grid=grid,
        in_specs=[xspec, xspec, xspec,
                  fullspec((1024, 2048)), fullspec((1024, 2048)), fullspec((1024, 2048)),
                  fullspec((1024, 48)), fullspec((1024, 128)),
                  fullspec((1024, 48)), fullspec((1024, 128)),
                  fullspec((1, 48)), fullspec((1, 128))],
        out_specs=[pl.BlockSpec((_TILE, 48), lambda i: (i, 0)),
                   pl.BlockSpec((_TILE, 128), lambda i: (i, 0))],
        out_shape=[jax.ShapeDtypeStruct((T, 48), jnp.float32),
                   jax.ShapeDtypeStruct((T, 128), jnp.float32)],
    )(x0, x1, x2, w0, w1, w2, wxg, wxl, mg, ml, cg, cl)


def kernel(frame_feature, Wg_d, bg_d, Wg_1, bg_1, Wg_o, bg_o,
           Wl_d, bl_d, Wl_1, bl_1, Wl_o, bl_o):
    T, B, C = frame_feature.shape
    half = _NW // 2
    nw1 = _NW + 1

    xt = frame_feature[:, 0, :]
    xp = jnp.pad(xt, ((1, 1), (0, 0)))
    xb = [xp[k:T + k].astype(jnp.bfloat16) for k in range(3)]

    wtaps = [jnp.concatenate(
        [jnp.transpose(Wg_d[:, :, k], (1, 0)), jnp.transpose(Wl_d[:, :, k], (1, 0))],
        axis=1).astype(jnp.bfloat16) for k in range(3)]
    bd_cat = jnp.concatenate([bg_d, bl_d])

    Mg = jnp.transpose(jnp.einsum('oc,cd->od', Wg_o[:, :, 0], Wg_1[:, :, 0]), (1, 0))
    Ml = jnp.transpose(jnp.einsum('oc,cd->od', Wl_o[:, :, 0], Wl_1[:, :, 0]), (1, 0))
    Wxg = jnp.transpose(Wg_o[:, :, 0], (1, 0))
    Wxl = jnp.transpose(Wl_o[:, :, 0], (1, 0))
    cg = (Wg_o[:, :, 0] @ bg_1 + bg_o)[None, :]
    cl = (Wl_o[:, :, 0] @ bl_1 + bl_o)[None, :]

    onescol = jnp.ones((T, 1), dtype=jnp.bfloat16)
    pad_rows = 8
    x1a = jnp.concatenate([xb[1], onescol,
                           jnp.zeros((T, pad_rows - 1), dtype=jnp.bfloat16)], axis=1)
    w1a = jnp.concatenate([wtaps[1], bd_cat[None, :].astype(jnp.bfloat16),
                           jnp.zeros((pad_rows - 1, 2048), dtype=jnp.bfloat16)], axis=0)
    Wxg_a = jnp.concatenate([Wxg, jnp.zeros((pad_rows, 48), Wxg.dtype)], axis=0)
    Wxl65_a = jnp.concatenate([Wxl, jnp.zeros((pad_rows, 65), Wxl.dtype)], axis=0)

    def pad65(a):
        return jnp.concatenate([a, jnp.zeros((a.shape[0], 128 - 65), a.dtype)], axis=1)

    gl, ll128 = _run_heads(
        jnp.concatenate([xb[0], jnp.zeros((T, pad_rows), jnp.bfloat16)], axis=1),
        x1a,
        jnp.concatenate([xb[2], jnp.zeros((T, pad_rows), jnp.bfloat16)], axis=1),
        jnp.concatenate([wtaps[0], jnp.zeros((pad_rows, 2048), jnp.bfloat16)], axis=0),
        w1a,
        jnp.concatenate([wtaps[2], jnp.zeros((pad_rows, 2048), jnp.bfloat16)], axis=0),
        Wxg_a.astype(jnp.bfloat16), pad65(Wxl65_a).astype(jnp.bfloat16),
        jnp.concatenate([Mg, jnp.zeros((pad_rows, 48), Mg.dtype)], axis=0).astype(jnp.bfloat16),
        pad65(jnp.concatenate([Ml, jnp.zeros((pad_rows, 65), Ml.dtype)], axis=0)).astype(jnp.bfloat16),
        cg.astype(jnp.float32), pad65(cl).astype(jnp.float32), T)
    ll = ll128[:, :65]

    g_logits = gl[:, None, :]
    l_logits = ll[:, None, :]

    x = jnp.pad(jnp.transpose(g_logits, (1, 2, 0)), ((0, 0), (0, 0), (half, half)))
    idx = jnp.arange(T)[:, None] + jnp.arange(nw1)[None, :]
    g_win = jnp.transpose(x[:, :, idx], (0, 2, 1, 3))
    l_win = jnp.transpose(l_logits[:, :, None, :], (1, 0, 2, 3))
    pred_scores = jax.nn.softmax(g_win + l_win, axis=-1)
    pred_scores = jnp.where(jnp.isnan(pred_scores), 0.0, pred_scores)
    padded = jnp.pad(pred_scores, ((0, 0), (half, half), (0, 0), (0, 0)))
    final = jnp.zeros_like(padded[:, :T])
    for i in range(_NW):
        final = final + padded[:, i:i + T]
    final = final.sum(axis=-1)
    phase = jnp.max(final, axis=-1)
    k = min(_NTOKEN, T)
    top_v, top_i = jax.lax.top_k(phase, k)
    gathered = frame_feature[top_i[0]]
    return gathered, top_i
